# baseline (speedup 1.0000x reference)
import functools

import jax
import jax.numpy as jnp
from jax import lax
from jax.experimental import pallas as pl
from jax.experimental.pallas import tpu as pltpu
from jax.experimental.pallas import tpu_sc as plsc

_MAX_CTX = 32768
_D = 1024

_NC, _NS = 2, 16
_NW = _NC * _NS
_SC_ROWS = 14336
_ROWS_PER_W = _SC_ROWS // _NW
_NBUF = 4
_CHUNK = 16
_NCHUNK = _ROWS_PER_W // _CHUNK
assert _NCHUNK % _NBUF == 0

_mesh = plsc.VectorSubcoreMesh(core_axis_name="c", subcore_axis_name="s")


@functools.partial(
    pl.kernel,
    out_type=jax.ShapeDtypeStruct((_SC_ROWS, _D), jnp.float32),
    mesh=_mesh,
    scratch_types=[
        pltpu.VMEM((_NBUF, _CHUNK, _D), jnp.float32),
        pltpu.SemaphoreType.DMA((_NBUF,)),
        pltpu.SemaphoreType.DMA((_NBUF,)),
    ],
)
def _sc_copy(src_hbm, out_hbm, buf, rsem, wsem):
    wid = lax.axis_index("s") * _NC + lax.axis_index("c")
    base = wid * _ROWS_PER_W
    ngroups = _NCHUNK // _NBUF
    for g in range(ngroups):
        for b in range(_NBUF):
            c = base + (g * _NBUF + b) * _CHUNK
            if g > 0:
                pltpu.make_async_copy(
                    buf.at[b], out_hbm.at[pl.ds(c - _NBUF * _CHUNK, _CHUNK)], wsem.at[b]
                ).wait()
            pltpu.make_async_copy(
                src_hbm.at[pl.ds(c, _CHUNK)], buf.at[b], rsem.at[b]
            ).start()
        for b in range(_NBUF):
            c = base + (g * _NBUF + b) * _CHUNK
            pltpu.make_async_copy(
                src_hbm.at[pl.ds(c, _CHUNK)], buf.at[b], rsem.at[b]
            ).wait()
            pltpu.make_async_copy(
                buf.at[b], out_hbm.at[pl.ds(c, _CHUNK)], wsem.at[b]
            ).start()
    for b in range(_NBUF):
        c = base + ((_NCHUNK - _NBUF) + b) * _CHUNK
        pltpu.make_async_copy(
            buf.at[b], out_hbm.at[pl.ds(c, _CHUNK)], wsem.at[b]
        ).wait()


_TC_ROWS = _MAX_CTX - _SC_ROWS
_TC_BLOCK = 1024
assert _TC_ROWS % _TC_BLOCK == 0


def _tc_copy_kernel(src_ref, dst_ref):
    dst_ref[...] = src_ref[...]


def kernel(h, mem):
    b, l, d = h.shape
    assert b * l == _MAX_CTX and d == _D
    flat = h.reshape(b * l, d)
    sc_part = _sc_copy(flat)
    tc_part = pl.pallas_call(
        _tc_copy_kernel,
        grid=(_TC_ROWS // _TC_BLOCK,),
        in_specs=[
            pl.BlockSpec((_TC_BLOCK, d), lambda i: (i + _SC_ROWS // _TC_BLOCK, 0))
        ],
        out_specs=pl.BlockSpec((_TC_BLOCK, d), lambda i: (i, 0)),
        out_shape=jax.ShapeDtypeStruct((_TC_ROWS, d), h.dtype),
    )(flat)
    new_mem = jnp.concatenate([sc_part, tc_part], axis=0)
    return (h, new_mem)


# speedup vs baseline: 1.6204x; 1.6204x over previous
import jax
import jax.numpy as jnp
from jax.experimental import pallas as pl
from jax.experimental.pallas import tpu as pltpu

_MAX_CTX = 32768
_D = 1024
_NBUF = 8
_CHUNK_ROWS = 1024
_NCHUNKS = _MAX_CTX // _CHUNK_ROWS
assert _NCHUNKS % _NBUF == 0


def _copy_kernel(src_ref, dst_ref, buf, rsem, wsem):
    ngroups = _NCHUNKS // _NBUF
    for g in range(ngroups):
        for b in range(_NBUF):
            c = g * _NBUF + b
            if g > 0:
                pltpu.make_async_copy(
                    buf.at[b], dst_ref.at[pl.ds((c - _NBUF) * _CHUNK_ROWS, _CHUNK_ROWS), :], wsem.at[b]
                ).wait()
            pltpu.make_async_copy(
                src_ref.at[pl.ds(c * _CHUNK_ROWS, _CHUNK_ROWS), :], buf.at[b], rsem.at[b]
            ).start()
        for b in range(_NBUF):
            c = g * _NBUF + b
            pltpu.make_async_copy(
                src_ref.at[pl.ds(c * _CHUNK_ROWS, _CHUNK_ROWS), :], buf.at[b], rsem.at[b]
            ).wait()
            pltpu.make_async_copy(
                buf.at[b], dst_ref.at[pl.ds(c * _CHUNK_ROWS, _CHUNK_ROWS), :], wsem.at[b]
            ).start()
    g = ngroups - 1
    for b in range(_NBUF):
        c = g * _NBUF + b
        pltpu.make_async_copy(
            buf.at[b], dst_ref.at[pl.ds(c * _CHUNK_ROWS, _CHUNK_ROWS), :], wsem.at[b]
        ).wait()


def kernel(h, mem):
    b, l, d = h.shape
    assert b * l == _MAX_CTX and d == _D
    flat = h.reshape(b * l, d)
    new_mem = pl.pallas_call(
        _copy_kernel,
        in_specs=[pl.BlockSpec(memory_space=pl.ANY)],
        out_specs=pl.BlockSpec(memory_space=pl.ANY),
        out_shape=jax.ShapeDtypeStruct((b * l, d), h.dtype),
        scratch_shapes=[
            pltpu.VMEM((_NBUF, _CHUNK_ROWS, _D), h.dtype),
            pltpu.SemaphoreType.DMA((_NBUF,)),
            pltpu.SemaphoreType.DMA((_NBUF,)),
        ],
    )(flat)
    return (h, new_mem)


# speedup vs baseline: 1.6269x; 1.0040x over previous
import jax
import jax.numpy as jnp
from jax.experimental import pallas as pl
from jax.experimental.pallas import tpu as pltpu

_MAX_CTX = 32768
_D = 1024
_NBUF = 4
_CHUNK_ROWS = 2048
_NCHUNKS = _MAX_CTX // _CHUNK_ROWS
assert _NCHUNKS % _NBUF == 0


def _copy_kernel(src_ref, dst_ref, buf, rsem, wsem):
    ngroups = _NCHUNKS // _NBUF
    for g in range(ngroups):
        for b in range(_NBUF):
            c = g * _NBUF + b
            if g > 0:
                pltpu.make_async_copy(
                    buf.at[b], dst_ref.at[pl.ds((c - _NBUF) * _CHUNK_ROWS, _CHUNK_ROWS), :], wsem.at[b]
                ).wait()
            pltpu.make_async_copy(
                src_ref.at[pl.ds(c * _CHUNK_ROWS, _CHUNK_ROWS), :], buf.at[b], rsem.at[b]
            ).start()
        for b in range(_NBUF):
            c = g * _NBUF + b
            pltpu.make_async_copy(
                src_ref.at[pl.ds(c * _CHUNK_ROWS, _CHUNK_ROWS), :], buf.at[b], rsem.at[b]
            ).wait()
            pltpu.make_async_copy(
                buf.at[b], dst_ref.at[pl.ds(c * _CHUNK_ROWS, _CHUNK_ROWS), :], wsem.at[b]
            ).start()
    g = ngroups - 1
    for b in range(_NBUF):
        c = g * _NBUF + b
        pltpu.make_async_copy(
            buf.at[b], dst_ref.at[pl.ds(c * _CHUNK_ROWS, _CHUNK_ROWS), :], wsem.at[b]
        ).wait()


def kernel(h, mem):
    b, l, d = h.shape
    assert b * l == _MAX_CTX and d == _D
    flat = h.reshape(b * l, d)
    new_mem = pl.pallas_call(
        _copy_kernel,
        in_specs=[pl.BlockSpec(memory_space=pl.ANY)],
        out_specs=pl.BlockSpec(memory_space=pl.ANY),
        out_shape=jax.ShapeDtypeStruct((b * l, d), h.dtype),
        scratch_shapes=[
            pltpu.VMEM((_NBUF, _CHUNK_ROWS, _D), h.dtype),
            pltpu.SemaphoreType.DMA((_NBUF,)),
            pltpu.SemaphoreType.DMA((_NBUF,)),
        ],
    )(flat)
    return (h, new_mem)
